# baseline (speedup 1.0000x reference)
import functools

import jax
import jax.numpy as jnp
from jax import lax
from jax.experimental import pallas as pl
from jax.experimental.pallas import tpu as pltpu
from jax.experimental.pallas import tpu_sc as plsc

_NC = 2
_NS = 16
_NW = _NC * _NS
_L = 16
_CHUNK = 128


@functools.lru_cache(maxsize=None)
def _build(B, D, VU, VI):
    bpw = B // _NW
    nchunk = bpw // _CHUNK
    nblk = bpw // _L

    mesh = plsc.VectorSubcoreMesh(core_axis_name="c", subcore_axis_name="s")

    @functools.partial(
        pl.kernel,
        mesh=mesh,
        out_type=jax.ShapeDtypeStruct((B,), jnp.float32),
        compiler_params=pltpu.CompilerParams(
            needs_layout_passes=False, use_tc_tiling_on_sc=False),
        scratch_types=[
            pltpu.VMEM((nchunk, _CHUNK), jnp.int32),
            pltpu.VMEM((nchunk, _CHUNK), jnp.int32),
            pltpu.VMEM((bpw * D,), jnp.float32),
            pltpu.VMEM((bpw * D,), jnp.float32),
            pltpu.VMEM((D * _L,), jnp.float32),
            pltpu.VMEM((_L,), jnp.float32),
            pltpu.VMEM((bpw,), jnp.float32),
            pltpu.SemaphoreType.DMA,
        ],
    )
    def gmf(uid_hbm, iid_hbm, ut_hbm, it_hbm, w_hbm, b_hbm, out_hbm,
            uidx, iidx, u_t, i_t, w_v, b_v, acc, sem):
        wid = lax.axis_index("s") * _NC + lax.axis_index("c")
        base = wid * bpw

        pltpu.sync_copy(uid_hbm.at[wid], uidx)
        pltpu.sync_copy(iid_hbm.at[wid], iidx)
        pltpu.sync_copy(w_hbm, w_v)
        pltpu.sync_copy(b_hbm, b_v)

        def fire(d, carry):
            for c in range(nchunk):
                dst = pl.ds(d * bpw + c * _CHUNK, _CHUNK)
                pltpu.async_copy(ut_hbm.at[d].at[uidx.at[c]], u_t.at[dst], sem)
                pltpu.async_copy(it_hbm.at[d].at[iidx.at[c]], i_t.at[dst], sem)
            return carry

        lax.fori_loop(0, D, fire, 0)

        zero = jnp.zeros((_L,), jnp.float32)
        for t in range(nblk):
            acc[pl.ds(t * _L, _L)] = zero

        pltpu.make_async_copy(ut_hbm.at[0].at[pl.ds(0, bpw * D)], u_t, sem).wait()
        pltpu.make_async_copy(it_hbm.at[0].at[pl.ds(0, bpw * D)], i_t, sem).wait()

        def dim_body(d, carry):
            wd = w_v[pl.ds(d * _L, _L)]
            for t in range(nblk):
                off = pl.ds(d * bpw + t * _L, _L)
                au = u_t[off]
                ai = i_t[off]
                plsc.addupdate(acc.at[pl.ds(t * _L, _L)], au * ai * wd)
            return carry

        lax.fori_loop(0, D, dim_body, 0)

        bvec = b_v[...]
        for t in range(nblk):
            x = acc[pl.ds(t * _L, _L)] + bvec
            acc[pl.ds(t * _L, _L)] = 1.0 / (1.0 + jnp.exp(-x))
        pltpu.sync_copy(acc, out_hbm.at[pl.ds(base, bpw)])

    return gmf


def kernel(uid, iid, user_table, item_table, W, b):
    B = uid.shape[0]
    VU, D = user_table.shape
    VI = item_table.shape[0]
    gmf = _build(B, D, VU, VI)
    uid3 = uid.reshape(_NW, -1, _CHUNK)
    iid3 = iid.reshape(_NW, -1, _CHUNK)
    ut_t = user_table.T
    it_t = item_table.T
    w_splat = jnp.broadcast_to(W.reshape(D, 1), (D, _L)).reshape(-1)
    b_splat = jnp.broadcast_to(b.reshape(()), (_L,)).astype(jnp.float32)
    out = gmf(uid3, iid3, ut_t, it_t, w_splat, b_splat)
    return out.reshape(B, 1)


# speedup vs baseline: 27.5171x; 27.5171x over previous
import functools

import jax
import jax.numpy as jnp
from jax import lax
from jax.experimental import pallas as pl
from jax.experimental.pallas import tpu as pltpu
from jax.experimental.pallas import tpu_sc as plsc

_NC = 2
_NS = 16
_NW = _NC * _NS
_L = 16
_TR = 8
_NE = 16


@functools.lru_cache(maxsize=None)
def _build(B, D, V):
    bpw = B // _NW
    nchunk = bpw // _NE
    nblk = bpw // _L
    ng = D // _TR

    mesh = plsc.VectorSubcoreMesh(core_axis_name="c", subcore_axis_name="s")

    @functools.partial(
        pl.kernel,
        mesh=mesh,
        out_type=jax.ShapeDtypeStruct((B,), jnp.float32),
        compiler_params=pltpu.CompilerParams(needs_layout_passes=False),
        scratch_types=[
            pltpu.VMEM((bpw,), jnp.int32),
            pltpu.VMEM((bpw,), jnp.int32),
            pltpu.VMEM((ng, _TR, _NE * _L), jnp.float32),
            pltpu.VMEM((ng, _TR, _NE * _L), jnp.float32),
            pltpu.VMEM((D * _L,), jnp.float32),
            pltpu.VMEM((_L,), jnp.float32),
            pltpu.VMEM((bpw,), jnp.float32),
            pltpu.SemaphoreType.DMA,
        ],
    )
    def gmf(uid_hbm, iid_hbm, ut_hbm, it_hbm, w_hbm, b_hbm, out_hbm,
            uidx, iidx, u_s, i_s, w_v, b_v, acc, sem):
        wid = lax.axis_index("s") * _NC + lax.axis_index("c")
        base = wid * bpw

        pltpu.sync_copy(uid_hbm.at[pl.ds(base, bpw)], uidx)
        pltpu.sync_copy(iid_hbm.at[pl.ds(base, bpw)], iidx)
        pltpu.sync_copy(w_hbm, w_v)
        pltpu.sync_copy(b_hbm, b_v)

        zero = jnp.zeros((_L,), jnp.float32)
        for t in range(nblk):
            acc[pl.ds(t * _L, _L)] = zero

        iota = lax.iota(jnp.int32, _L)

        def chunk_body(c, carry):
            uv = uidx[pl.ds(c * _NE, _NE)]
            iv = iidx[pl.ds(c * _NE, _NE)]
            copies = []
            for l in range(_NE):
                ua = (uv[l] // _L) * _L
                ia = (iv[l] // _L) * _L
                for g in range(ng):
                    copies.append(pltpu.async_copy(
                        ut_hbm.at[g].at[pl.ds(0, _TR), pl.ds(ua, _L)],
                        u_s.at[g].at[pl.ds(0, _TR), pl.ds(l * _L, _L)], sem))
                    copies.append(pltpu.async_copy(
                        it_hbm.at[g].at[pl.ds(0, _TR), pl.ds(ia, _L)],
                        i_s.at[g].at[pl.ds(0, _TR), pl.ds(l * _L, _L)], sem))
            for cp in copies:
                cp.wait()

            uoff = iota * _L + jnp.bitwise_and(uv, _L - 1)
            ioff = iota * _L + jnp.bitwise_and(iv, _L - 1)

            asl = pl.ds(c * _NE, _L)
            a = acc[asl]
            for g in range(ng):
                for r in range(_TR):
                    d = g * _TR + r
                    wd = w_v[pl.ds(d * _L, _L)]
                    gu = plsc.load_gather(u_s, [jnp.full((_L,), g, jnp.int32),
                                                jnp.full((_L,), r, jnp.int32),
                                                uoff])
                    gi = plsc.load_gather(i_s, [jnp.full((_L,), g, jnp.int32),
                                                jnp.full((_L,), r, jnp.int32),
                                                ioff])
                    a = a + gu * gi * wd
            acc[asl] = a
            return carry

        lax.fori_loop(0, nchunk, chunk_body, 0)

        bvec = b_v[...]
        for t in range(nblk):
            x = acc[pl.ds(t * _L, _L)] + bvec
            acc[pl.ds(t * _L, _L)] = 1.0 / (1.0 + jnp.exp(-x))
        pltpu.sync_copy(acc, out_hbm.at[pl.ds(base, bpw)])

    return gmf


def kernel(uid, iid, user_table, item_table, W, b):
    B = uid.shape[0]
    V, D = user_table.shape
    gmf = _build(B, D, V)
    ut3 = user_table.T.reshape(D // _TR, _TR, V)
    it3 = item_table.T.reshape(D // _TR, _TR, V)
    w_splat = jnp.broadcast_to(W.reshape(D, 1), (D, _L)).reshape(-1)
    b_splat = jnp.broadcast_to(b.reshape(()), (_L,)).astype(jnp.float32)
    out = gmf(uid, iid, ut3, it3, w_splat, b_splat)
    return out.reshape(B, 1)


# speedup vs baseline: 65.0559x; 2.3642x over previous
import functools

import jax
import jax.numpy as jnp
from jax import lax
from jax.experimental import pallas as pl
from jax.experimental.pallas import tpu as pltpu
from jax.experimental.pallas import tpu_sc as plsc

_NC = 2
_NS = 16
_NW = _NC * _NS
_L = 16
_TR = 8
_NE = 16


@functools.lru_cache(maxsize=None)
def _build(B, D, V):
    bpw = B // _NW
    nchunk = bpw // _NE
    nblk = bpw // _L
    ng = D // _TR

    mesh = plsc.VectorSubcoreMesh(core_axis_name="c", subcore_axis_name="s")

    @functools.partial(
        pl.kernel,
        mesh=mesh,
        out_type=jax.ShapeDtypeStruct((B,), jnp.float32),
        compiler_params=pltpu.CompilerParams(needs_layout_passes=False),
        scratch_types=[
            pltpu.VMEM((bpw + _L,), jnp.int32),
            pltpu.VMEM((bpw + _L,), jnp.int32),
            pltpu.VMEM((2, ng, _TR, _NE * _L), jnp.float32),
            pltpu.VMEM((2, ng, _TR, _NE * _L), jnp.float32),
            pltpu.VMEM((D * _L,), jnp.float32),
            pltpu.VMEM((_L,), jnp.float32),
            pltpu.VMEM((bpw,), jnp.float32),
            pltpu.SemaphoreType.DMA,
            pltpu.SemaphoreType.DMA,
        ],
    )
    def gmf(uid_hbm, iid_hbm, ut_hbm, it_hbm, w_hbm, b_hbm, out_hbm,
            uidx, iidx, u_s, i_s, w_v, b_v, acc, sem0, sem1):
        wid = lax.axis_index("s") * _NC + lax.axis_index("c")
        base = wid * bpw

        pltpu.sync_copy(uid_hbm.at[pl.ds(base, bpw)], uidx.at[pl.ds(0, bpw)])
        pltpu.sync_copy(iid_hbm.at[pl.ds(base, bpw)], iidx.at[pl.ds(0, bpw)])
        pltpu.sync_copy(w_hbm, w_v)
        pltpu.sync_copy(b_hbm, b_v)

        zero = jnp.zeros((_L,), jnp.float32)
        for t in range(nblk):
            acc[pl.ds(t * _L, _L)] = zero
        uidx[pl.ds(bpw, _L)] = jnp.zeros((_L,), jnp.int32)
        iidx[pl.ds(bpw, _L)] = jnp.zeros((_L,), jnp.int32)

        iota = lax.iota(jnp.int32, _L)

        def fire(c, buf, sem):
            def one(l, carry):
                e = c * _NE + l
                u = uidx[pl.ds(e, _L)][0]
                i = iidx[pl.ds(e, _L)][0]
                ua = (u // _L) * _L
                ia = (i // _L) * _L
                dsl = pl.ds(l * _L, _L)
                for g in range(ng):
                    pltpu.async_copy(
                        ut_hbm.at[g].at[pl.ds(0, _TR), pl.ds(ua, _L)],
                        u_s.at[buf].at[g].at[pl.ds(0, _TR), dsl], sem)
                    pltpu.async_copy(
                        it_hbm.at[g].at[pl.ds(0, _TR), pl.ds(ia, _L)],
                        i_s.at[buf].at[g].at[pl.ds(0, _TR), dsl], sem)
                return carry
            lax.fori_loop(0, _NE, one, 0)

        csrc = ut_hbm.at[pl.ds(0, ng), pl.ds(0, _TR), pl.ds(0, _NE * _L)]

        def drain(buf, sem):
            pltpu.make_async_copy(csrc, u_s.at[buf], sem).wait()
            pltpu.make_async_copy(csrc, i_s.at[buf], sem).wait()

        def compute(c, buf):
            uv = uidx[pl.ds(c * _NE, _NE)]
            iv = iidx[pl.ds(c * _NE, _NE)]
            uoff = iota * _L + jnp.bitwise_and(uv, _L - 1)
            ioff = iota * _L + jnp.bitwise_and(iv, _L - 1)
            bsp = jnp.full((_L,), buf, jnp.int32)

            def dim_body(d, a):
                g = d // _TR
                r = d % _TR
                gsp = jnp.full((_L,), g, jnp.int32)
                rsp = jnp.full((_L,), r, jnp.int32)
                wd = w_v[pl.ds(d * _L, _L)]
                gu = plsc.load_gather(u_s, [bsp, gsp, rsp, uoff])
                gi = plsc.load_gather(i_s, [bsp, gsp, rsp, ioff])
                return a + gu * gi * wd

            asl = pl.ds(c * _NE, _L)
            acc[asl] = lax.fori_loop(0, D, dim_body, acc[asl])

        fire(0, 0, sem0)

        def pair_body(k, carry):
            c0 = 2 * k
            fire(c0 + 1, 1, sem1)
            drain(0, sem0)
            compute(c0, 0)
            fire(c0 + 2, 0, sem0)
            drain(1, sem1)
            compute(c0 + 1, 1)
            return carry

        lax.fori_loop(0, nchunk // 2 - 1, pair_body, 0)

        c_last = nchunk - 1
        fire(c_last, 1, sem1)
        drain(0, sem0)
        compute(c_last - 1, 0)
        drain(1, sem1)
        compute(c_last, 1)

        bvec = b_v[...]
        for t in range(nblk):
            x = acc[pl.ds(t * _L, _L)] + bvec
            acc[pl.ds(t * _L, _L)] = 1.0 / (1.0 + jnp.exp(-x))
        pltpu.sync_copy(acc, out_hbm.at[pl.ds(base, bpw)])

    return gmf


def kernel(uid, iid, user_table, item_table, W, b):
    B = uid.shape[0]
    V, D = user_table.shape
    gmf = _build(B, D, V)
    ut3 = user_table.T.reshape(D // _TR, _TR, V)
    it3 = item_table.T.reshape(D // _TR, _TR, V)
    w_splat = jnp.broadcast_to(W.reshape(D, 1), (D, _L)).reshape(-1)
    b_splat = jnp.broadcast_to(b.reshape(()), (_L,)).astype(jnp.float32)
    out = gmf(uid, iid, ut3, it3, w_splat, b_splat)
    return out.reshape(B, 1)
